# baseline (device time: 110457 ns/iter reference)
import jax
import jax.numpy as jnp
from jax import lax
from jax.experimental import pallas as pl
from jax.experimental.pallas import tpu as pltpu

N_DEV = 16
S = 2
HOPS = 8


def kernel(x, w_mat, scale_x, scale_w):
    m_per, k = x.shape
    n_per = w_mat.shape[1]
    m_glob = N_DEV * m_per
    mp = m_per // S

    def body(x_ref, w_ref, sx_ref, sw_ref, out_ref, gath_ref,
             cw_send, cw_recv, ccw_send, ccw_recv):
        me = lax.axis_index("i")
        left = lax.rem(me + N_DEV - 1, N_DEV)
        right = lax.rem(me + 1, N_DEV)

        barrier_sem = pltpu.get_barrier_semaphore()
        pl.semaphore_signal(barrier_sem, inc=1, device_id=(left,),
                            device_id_type=pl.DeviceIdType.MESH)
        pl.semaphore_signal(barrier_sem, inc=1, device_id=(right,),
                            device_id_type=pl.DeviceIdType.MESH)
        pl.semaphore_wait(barrier_sem, 2)

        scale = sx_ref[0] * sw_ref[0]
        w_bf = w_ref[:, :].astype(jnp.bfloat16)

        def compute_rows(slot, r0, nr):
            origin = lax.rem(me - slot + N_DEV, N_DEV)
            a = gath_ref[slot, pl.ds(r0, nr), :].astype(jnp.bfloat16)
            acc = jnp.dot(a, w_bf, preferred_element_type=jnp.float32)
            out_ref[pl.ds(origin * m_per + r0, nr), :] = (
                jnp.maximum(acc * scale, 0.0))

        def compute(slot):
            compute_rows(slot, 0, m_per)

        def mk_cw(h, p):
            return pltpu.make_async_remote_copy(
                src_ref=gath_ref.at[h, pl.ds(p * mp, mp), :],
                dst_ref=gath_ref.at[h + 1, pl.ds(p * mp, mp), :],
                send_sem=cw_send.at[h, p],
                recv_sem=cw_recv.at[h, p],
                device_id=(right,),
                device_id_type=pl.DeviceIdType.MESH,
            )

        def mk_ccw(h, p):
            return pltpu.make_async_remote_copy(
                src_ref=gath_ref.at[(N_DEV - h) % N_DEV, pl.ds(p * mp, mp), :],
                dst_ref=gath_ref.at[N_DEV - 1 - h, pl.ds(p * mp, mp), :],
                send_sem=ccw_send.at[h, p],
                recv_sem=ccw_recv.at[h, p],
                device_id=(left,),
                device_id_type=pl.DeviceIdType.MESH,
            )

        cw = [[None] * S for _ in range(HOPS)]
        ccw = [[None] * S for _ in range(HOPS)]

        def mk_dir(h, p, r0, nr, cwdir):
            src_slot = h if cwdir else (N_DEV - h) % N_DEV
            dst_slot = h + 1 if cwdir else N_DEV - 1 - h
            return pltpu.make_async_remote_copy(
                src_ref=gath_ref.at[src_slot, pl.ds(r0, nr), :],
                dst_ref=gath_ref.at[dst_slot, pl.ds(r0, nr), :],
                send_sem=(cw_send if cwdir else ccw_send).at[h, p],
                recv_sem=(cw_recv if cwdir else ccw_recv).at[h, p],
                device_id=(right if cwdir else left,),
                device_id_type=pl.DeviceIdType.MESH,
            )

        for h in range(HOPS):
            if h == 0:
                gath_ref[0, :, :] = x_ref[:, :].astype(jnp.float8_e4m3fn)
                cw[0][0] = mk_dir(0, 0, 0, m_per, True)
                cw[0][0].start()
                ccw[0][0] = mk_dir(0, 0, 0, m_per, False)
                ccw[0][0].start()
            else:
                cw[h - 1][0].wait_recv()
                if h < HOPS - 1:
                    cw[h][0] = mk_dir(h, 0, 0, m_per, True)
                else:
                    cw[h][0] = mk_dir(h, 0, 0, m_per // 2, True)
                cw[h][0].start()
                ccw[h - 1][0].wait_recv()
                if h < HOPS - 1:
                    ccw[h][0] = mk_dir(h, 0, 0, m_per, False)
                else:
                    ccw[h][1] = mk_dir(h, 1, m_per // 2, m_per // 2, False)
                    ccw[h][0] = None
                if h < HOPS - 1:
                    ccw[h][0].start()
                else:
                    ccw[h][1].start()
            if h == 0:
                compute(0)
            else:
                compute(h)
                compute(N_DEV - h)
        for p in range(S // 2):
            cw[HOPS - 1][p].wait_recv()
        compute_rows(HOPS, 0, m_per // 2)
        for p in range(S // 2, S):
            ccw[HOPS - 1][p].wait_recv()
        compute_rows(HOPS, m_per // 2, m_per // 2)

        for hh in cw:
            for r in hh:
                if r is not None:
                    r.wait_send()
        for hh in ccw:
            for r in hh:
                if r is not None:
                    r.wait_send()

    return pl.pallas_call(
        body,
        out_shape=jax.ShapeDtypeStruct((m_glob, n_per), jnp.float32),
        in_specs=[
            pl.BlockSpec(memory_space=pltpu.VMEM),
            pl.BlockSpec(memory_space=pltpu.VMEM),
            pl.BlockSpec(memory_space=pltpu.SMEM),
            pl.BlockSpec(memory_space=pltpu.SMEM),
        ],
        out_specs=pl.BlockSpec(memory_space=pltpu.VMEM),
        scratch_shapes=[
            pltpu.VMEM((N_DEV, m_per, k), jnp.float8_e4m3fn),
            pltpu.SemaphoreType.DMA((HOPS, S)),
            pltpu.SemaphoreType.DMA((HOPS, S)),
            pltpu.SemaphoreType.DMA((HOPS, S)),
            pltpu.SemaphoreType.DMA((HOPS, S)),
        ],
        compiler_params=pltpu.CompilerParams(collective_id=0),
    )(x, w_mat, scale_x, scale_w)


# device time: 100587 ns/iter; 1.0981x vs baseline; 1.0981x over previous
import jax
import jax.numpy as jnp
from jax import lax
from jax.experimental import pallas as pl
from jax.experimental.pallas import tpu as pltpu

N_DEV = 16
S = 2
HOPS = 8

HPERM = (0, 4, 8, 12, 13, 9, 5, 1, 2, 6, 10, 14, 15, 11, 7, 3)
HINV = (0, 7, 8, 15, 1, 6, 9, 14, 2, 5, 10, 13, 3, 4, 11, 12)


def kernel(x, w_mat, scale_x, scale_w):
    m_per, k = x.shape
    n_per = w_mat.shape[1]
    m_glob = N_DEV * m_per
    mp = m_per // S

    def body(x_ref, w_ref, sx_ref, sw_ref, hperm_ref, hinv_ref,
             out_ref, gath_ref, cw_send, cw_recv, ccw_send, ccw_recv):
        me = lax.axis_index("i")
        pos = hinv_ref[me]
        left = hperm_ref[lax.rem(pos + N_DEV - 1, N_DEV)]
        right = hperm_ref[lax.rem(pos + 1, N_DEV)]

        barrier_sem = pltpu.get_barrier_semaphore()
        pl.semaphore_signal(barrier_sem, inc=1, device_id=(left,),
                            device_id_type=pl.DeviceIdType.MESH)
        pl.semaphore_signal(barrier_sem, inc=1, device_id=(right,),
                            device_id_type=pl.DeviceIdType.MESH)
        pl.semaphore_wait(barrier_sem, 2)

        scale = sx_ref[0] * sw_ref[0]
        w_bf = w_ref[:, :].astype(jnp.bfloat16)

        def compute_rows(slot, r0, nr):
            origin = hperm_ref[lax.rem(pos - slot + N_DEV, N_DEV)]
            a = gath_ref[slot, pl.ds(r0, nr), :].astype(jnp.bfloat16)
            acc = jnp.dot(a, w_bf, preferred_element_type=jnp.float32)
            out_ref[pl.ds(origin * m_per + r0, nr), :] = (
                jnp.maximum(acc * scale, 0.0))

        def compute(slot):
            compute_rows(slot, 0, m_per)

        def mk_cw(h, p):
            return pltpu.make_async_remote_copy(
                src_ref=gath_ref.at[h, pl.ds(p * mp, mp), :],
                dst_ref=gath_ref.at[h + 1, pl.ds(p * mp, mp), :],
                send_sem=cw_send.at[h, p],
                recv_sem=cw_recv.at[h, p],
                device_id=(right,),
                device_id_type=pl.DeviceIdType.MESH,
            )

        def mk_ccw(h, p):
            return pltpu.make_async_remote_copy(
                src_ref=gath_ref.at[(N_DEV - h) % N_DEV, pl.ds(p * mp, mp), :],
                dst_ref=gath_ref.at[N_DEV - 1 - h, pl.ds(p * mp, mp), :],
                send_sem=ccw_send.at[h, p],
                recv_sem=ccw_recv.at[h, p],
                device_id=(left,),
                device_id_type=pl.DeviceIdType.MESH,
            )

        cw = [[None] * S for _ in range(HOPS)]
        ccw = [[None] * S for _ in range(HOPS)]

        for h in range(HOPS):
            if h == 0:
                gath_ref[0, pl.ds(0, mp), :] = (
                    x_ref[pl.ds(0, mp), :].astype(jnp.float8_e4m3fn))
                cw[0][0] = mk_cw(0, 0)
                cw[0][0].start()
                ccw[0][0] = mk_ccw(0, 0)
                ccw[0][0].start()
                gath_ref[0, pl.ds(mp, mp), :] = (
                    x_ref[pl.ds(mp, mp), :].astype(jnp.float8_e4m3fn))
                cw[0][1] = mk_cw(0, 1)
                cw[0][1].start()
                ccw[0][1] = mk_ccw(0, 1)
                ccw[0][1].start()
            else:
                for p in range(S):
                    cw[h - 1][p].wait_recv()
                    if h < HOPS - 1 or p < S // 2:
                        cw[h][p] = mk_cw(h, p)
                        cw[h][p].start()
                for p in range(S):
                    ccw[h - 1][p].wait_recv()
                    if h < HOPS - 1 or p >= S // 2:
                        ccw[h][p] = mk_ccw(h, p)
                        ccw[h][p].start()
            if h == 0:
                compute(0)
            else:
                compute(h)
                compute(N_DEV - h)
        for p in range(S // 2):
            cw[HOPS - 1][p].wait_recv()
        compute_rows(HOPS, 0, m_per // 2)
        for p in range(S // 2, S):
            ccw[HOPS - 1][p].wait_recv()
        compute_rows(HOPS, m_per // 2, m_per // 2)

        for hh in cw:
            for r in hh:
                if r is not None:
                    r.wait_send()
        for hh in ccw:
            for r in hh:
                if r is not None:
                    r.wait_send()

    return pl.pallas_call(
        body,
        out_shape=jax.ShapeDtypeStruct((m_glob, n_per), jnp.float32),
        in_specs=[
            pl.BlockSpec(memory_space=pltpu.VMEM),
            pl.BlockSpec(memory_space=pltpu.VMEM),
            pl.BlockSpec(memory_space=pltpu.SMEM),
            pl.BlockSpec(memory_space=pltpu.SMEM),
            pl.BlockSpec(memory_space=pltpu.SMEM),
            pl.BlockSpec(memory_space=pltpu.SMEM),
        ],
        out_specs=pl.BlockSpec(memory_space=pltpu.VMEM),
        scratch_shapes=[
            pltpu.VMEM((N_DEV, m_per, k), jnp.float8_e4m3fn),
            pltpu.SemaphoreType.DMA((HOPS, S)),
            pltpu.SemaphoreType.DMA((HOPS, S)),
            pltpu.SemaphoreType.DMA((HOPS, S)),
            pltpu.SemaphoreType.DMA((HOPS, S)),
        ],
        compiler_params=pltpu.CompilerParams(collective_id=0),
    )(x, w_mat, scale_x, scale_w,
      jnp.array(HPERM, dtype=jnp.int32), jnp.array(HINV, dtype=jnp.int32))


# device time: 97674 ns/iter; 1.1309x vs baseline; 1.0298x over previous
import jax
import jax.numpy as jnp
from jax import lax
from jax.experimental import pallas as pl
from jax.experimental.pallas import tpu as pltpu

N_DEV = 16
S = 2
HOPS = 8


def kernel(x, w_mat, scale_x, scale_w):
    m_per, k = x.shape
    n_per = w_mat.shape[1]
    m_glob = N_DEV * m_per
    mp = m_per // S

    def body(x_ref, w_ref, sx_ref, sw_ref, out_ref, gath_ref,
             cw_send, cw_recv, ccw_send, ccw_recv):
        me = lax.axis_index("i")
        left = lax.rem(me + N_DEV - 1, N_DEV)
        right = lax.rem(me + 1, N_DEV)

        barrier_sem = pltpu.get_barrier_semaphore()
        pl.semaphore_signal(barrier_sem, inc=1, device_id=(left,),
                            device_id_type=pl.DeviceIdType.MESH)
        pl.semaphore_signal(barrier_sem, inc=1, device_id=(right,),
                            device_id_type=pl.DeviceIdType.MESH)
        pl.semaphore_wait(barrier_sem, 2)

        scale = sx_ref[0] * sw_ref[0]
        w_bf = w_ref[:, :].astype(jnp.bfloat16)

        def compute_rows(slot, r0, nr):
            origin = lax.rem(me - slot + N_DEV, N_DEV)
            a = gath_ref[slot, pl.ds(r0, nr), :].astype(jnp.bfloat16)
            acc = jnp.dot(a, w_bf, preferred_element_type=jnp.float32)
            out_ref[pl.ds(origin * m_per + r0, nr), :] = (
                jnp.maximum(acc * scale, 0.0))

        def compute(slot):
            compute_rows(slot, 0, m_per)

        def mk_cw(h, p):
            return pltpu.make_async_remote_copy(
                src_ref=gath_ref.at[h, pl.ds(p * mp, mp), :],
                dst_ref=gath_ref.at[h + 1, pl.ds(p * mp, mp), :],
                send_sem=cw_send.at[h, p],
                recv_sem=cw_recv.at[h, p],
                device_id=(right,),
                device_id_type=pl.DeviceIdType.MESH,
            )

        def mk_ccw(h, p):
            return pltpu.make_async_remote_copy(
                src_ref=gath_ref.at[(N_DEV - h) % N_DEV, pl.ds(p * mp, mp), :],
                dst_ref=gath_ref.at[N_DEV - 1 - h, pl.ds(p * mp, mp), :],
                send_sem=ccw_send.at[h, p],
                recv_sem=ccw_recv.at[h, p],
                device_id=(left,),
                device_id_type=pl.DeviceIdType.MESH,
            )

        cw = [[None] * S for _ in range(HOPS)]
        ccw = [[None] * S for _ in range(HOPS)]

        for h in range(HOPS):
            if h == 0:
                gath_ref[0, pl.ds(0, mp), :] = (
                    x_ref[pl.ds(0, mp), :].astype(jnp.float8_e4m3fn))
                cw[0][0] = mk_cw(0, 0)
                cw[0][0].start()
                ccw[0][0] = mk_ccw(0, 0)
                ccw[0][0].start()
                gath_ref[0, pl.ds(mp, mp), :] = (
                    x_ref[pl.ds(mp, mp), :].astype(jnp.float8_e4m3fn))
                cw[0][1] = mk_cw(0, 1)
                cw[0][1].start()
                ccw[0][1] = mk_ccw(0, 1)
                ccw[0][1].start()
            else:
                for p in range(S):
                    cw[h - 1][p].wait_recv()
                    if h < HOPS - 1 or p < S // 2:
                        cw[h][p] = mk_cw(h, p)
                        cw[h][p].start()
                for p in range(S):
                    ccw[h - 1][p].wait_recv()
                    if h < HOPS - 1 or p >= S // 2:
                        ccw[h][p] = mk_ccw(h, p)
                        ccw[h][p].start()
            if h == 0:
                compute(0)
            else:
                compute(h)
                compute(N_DEV - h)
        for p in range(S // 2):
            cw[HOPS - 1][p].wait_recv()
        compute_rows(HOPS, 0, m_per // 2)
        for p in range(S // 2, S):
            ccw[HOPS - 1][p].wait_recv()
        compute_rows(HOPS, m_per // 2, m_per // 2)

        for hh in cw:
            for r in hh:
                if r is not None:
                    r.wait_send()
        for hh in ccw:
            for r in hh:
                if r is not None:
                    r.wait_send()

    return pl.pallas_call(
        body,
        out_shape=jax.ShapeDtypeStruct((m_glob, n_per), jnp.float32),
        in_specs=[
            pl.BlockSpec(memory_space=pltpu.VMEM),
            pl.BlockSpec(memory_space=pltpu.VMEM),
            pl.BlockSpec(memory_space=pltpu.SMEM),
            pl.BlockSpec(memory_space=pltpu.SMEM),
        ],
        out_specs=pl.BlockSpec(memory_space=pltpu.VMEM),
        scratch_shapes=[
            pltpu.VMEM((N_DEV, m_per, k), jnp.float8_e4m3fn),
            pltpu.SemaphoreType.DMA((HOPS, S)),
            pltpu.SemaphoreType.DMA((HOPS, S)),
            pltpu.SemaphoreType.DMA((HOPS, S)),
            pltpu.SemaphoreType.DMA((HOPS, S)),
        ],
        compiler_params=pltpu.CompilerParams(collective_id=0),
    )(x, w_mat, scale_x, scale_w)
